# baseline (speedup 1.0000x reference)
import functools

import jax
import jax.numpy as jnp
from jax import lax
from jax.experimental import pallas as pl
from jax.experimental.pallas import tpu as pltpu
from jax.experimental.pallas import tpu_sc as plsc

BATCH = 16384
EMB_DIM = 32
LANES = 16
PACK = 128 // EMB_DIM
NUM_WORKERS = 32
B_PER_W = BATCH // NUM_WORKERS
CHUNK = 256
N_CHUNKS = B_PER_W // CHUNK
C_GROUPS = CHUNK // LANES


def _dot_sigmoid_kernel(user_ids_hbm, item_ids_hbm, user_emb_hbm,
                        item_emb_hbm, out_hbm,
                        uidx_v, iidx_v, usup_v, isup_v, urows_v, irows_v,
                        out_v, sem):
    nc = 2
    wid = lax.axis_index("s") * nc + lax.axis_index("c")
    base = wid * B_PER_W

    pltpu.sync_copy(user_ids_hbm.at[pl.ds(base, B_PER_W)], uidx_v)
    pltpu.sync_copy(item_ids_hbm.at[pl.ds(base, B_PER_W)], iidx_v)

    lane_iota = lax.iota(jnp.int32, LANES)

    for c in range(N_CHUNKS):
        def sup_body(j, carry):
            off = c * CHUNK + j * LANES
            usup_v[pl.ds(j * LANES, LANES)] = (
                uidx_v[pl.ds(off, LANES)] >> 2)
            isup_v[pl.ds(j * LANES, LANES)] = (
                iidx_v[pl.ds(off, LANES)] >> 2)
            return carry

        lax.fori_loop(0, C_GROUPS, sup_body, 0)

        cp_u = pltpu.make_async_copy(user_emb_hbm.at[usup_v], urows_v, sem)
        cp_i = pltpu.make_async_copy(item_emb_hbm.at[isup_v], irows_v, sem)
        cp_u.start()
        cp_i.start()
        cp_u.wait()
        cp_i.wait()

        def group_body(g, carry):
            off = c * CHUNK + g * LANES
            uids = uidx_v[pl.ds(off, LANES)]
            iids = iidx_v[pl.ds(off, LANES)]
            ucol = (uids & 3) << 5
            icol = (iids & 3) << 5
            row_idx = g * LANES + lane_iota
            acc = jnp.zeros((LANES,), jnp.float32)
            for k in range(EMB_DIM):
                uv = plsc.load_gather(urows_v, [row_idx, ucol + k])
                iv = plsc.load_gather(irows_v, [row_idx, icol + k])
                acc = acc + uv * iv
            sig = 1.0 / (1.0 + jnp.exp(-acc))
            out_v[pl.ds(off, LANES)] = sig
            return carry

        lax.fori_loop(0, C_GROUPS, group_body, 0)

    pltpu.sync_copy(out_v, out_hbm.at[pl.ds(base, B_PER_W)])


@jax.jit
def _run(user_ids, item_ids, user_emb, item_emb):
    user_sup = user_emb.reshape(-1, PACK * EMB_DIM)
    item_sup = item_emb.reshape(-1, PACK * EMB_DIM)
    mesh = plsc.VectorSubcoreMesh(core_axis_name="c", subcore_axis_name="s")
    kfn = functools.partial(
        pl.kernel,
        mesh=mesh,
        out_type=jax.ShapeDtypeStruct((BATCH,), jnp.float32),
        scratch_types=[
            pltpu.VMEM((B_PER_W,), jnp.int32),
            pltpu.VMEM((B_PER_W,), jnp.int32),
            pltpu.VMEM((CHUNK,), jnp.int32),
            pltpu.VMEM((CHUNK,), jnp.int32),
            pltpu.VMEM((CHUNK, PACK * EMB_DIM), jnp.float32),
            pltpu.VMEM((CHUNK, PACK * EMB_DIM), jnp.float32),
            pltpu.VMEM((B_PER_W,), jnp.float32),
            pltpu.SemaphoreType.DMA,
        ],
        compiler_params=pltpu.CompilerParams(needs_layout_passes=False),
    )(_dot_sigmoid_kernel)
    return kfn(user_ids, item_ids, user_sup, item_sup)


def kernel(user_ids, item_ids, user_emb, item_emb):
    return _run(user_ids.astype(jnp.int32), item_ids.astype(jnp.int32),
                user_emb, item_emb)


# speedup vs baseline: 1.3574x; 1.3574x over previous
import functools

import jax
import jax.numpy as jnp
from jax import lax
from jax.experimental import pallas as pl
from jax.experimental.pallas import tpu as pltpu
from jax.experimental.pallas import tpu_sc as plsc

BATCH = 16384
EMB_DIM = 32
LANES = 16
ROWG = 8
NUM_WORKERS = 32
B_PER_W = BATCH // NUM_WORKERS
N_CHUNKS = B_PER_W // LANES


def _dot_sigmoid_kernel(user_ids_hbm, item_ids_hbm, user_emb_hbm,
                        item_emb_hbm, out_hbm,
                        uidx_v, iidx_v, ubuf_v, ibuf_v,
                        out_v, sem_u, sem_i):
    nc = 2
    wid = lax.axis_index("s") * nc + lax.axis_index("c")
    base = wid * B_PER_W

    pltpu.sync_copy(user_ids_hbm.at[pl.ds(base, B_PER_W)], uidx_v)
    pltpu.sync_copy(item_ids_hbm.at[pl.ds(base, B_PER_W)], iidx_v)
    lane_iota = lax.iota(jnp.int32, LANES)

    def chunk_body(c, carry):
        off = c * LANES
        uvec = uidx_v[pl.ds(off, LANES)]
        ivec = iidx_v[pl.ds(off, LANES)]
        ubase = uvec & ~(ROWG - 1)
        ibase = ivec & ~(ROWG - 1)
        copies = []
        for j in range(LANES):
            urow = pl.multiple_of(ubase[j], ROWG)
            irow = pl.multiple_of(ibase[j], ROWG)
            cp_u = pltpu.make_async_copy(
                user_emb_hbm.at[pl.ds(urow, ROWG), :],
                ubuf_v.at[pl.ds(j * ROWG, ROWG), :], sem_u)
            cp_i = pltpu.make_async_copy(
                item_emb_hbm.at[pl.ds(irow, ROWG), :],
                ibuf_v.at[pl.ds(j * ROWG, ROWG), :], sem_i)
            cp_u.start()
            cp_i.start()
            copies.append((cp_u, cp_i))
        for cp_u, cp_i in copies:
            cp_u.wait()
            cp_i.wait()

        urow_idx = lane_iota * ROWG + (uvec & (ROWG - 1))
        irow_idx = lane_iota * ROWG + (ivec & (ROWG - 1))
        acc = jnp.zeros((LANES,), jnp.float32)
        for k in range(EMB_DIM):
            kcol = jnp.full((LANES,), k, jnp.int32)
            uv = plsc.load_gather(ubuf_v, [urow_idx, kcol])
            iv = plsc.load_gather(ibuf_v, [irow_idx, kcol])
            acc = acc + uv * iv
        sig = 1.0 / (1.0 + jnp.exp(-acc))
        out_v[pl.ds(off, LANES)] = sig
        return carry

    lax.fori_loop(0, N_CHUNKS, chunk_body, 0)

    pltpu.sync_copy(out_v, out_hbm.at[pl.ds(base, B_PER_W)])


@jax.jit
def _run(user_ids, item_ids, user_emb, item_emb):
    mesh = plsc.VectorSubcoreMesh(core_axis_name="c", subcore_axis_name="s")
    kfn = functools.partial(
        pl.kernel,
        mesh=mesh,
        out_type=jax.ShapeDtypeStruct((BATCH,), jnp.float32),
        scratch_types=[
            pltpu.VMEM((B_PER_W,), jnp.int32),
            pltpu.VMEM((B_PER_W,), jnp.int32),
            pltpu.VMEM((LANES * ROWG, EMB_DIM), jnp.float32),
            pltpu.VMEM((LANES * ROWG, EMB_DIM), jnp.float32),
            pltpu.VMEM((B_PER_W,), jnp.float32),
            pltpu.SemaphoreType.DMA,
            pltpu.SemaphoreType.DMA,
        ],
        compiler_params=pltpu.CompilerParams(needs_layout_passes=False),
    )(_dot_sigmoid_kernel)
    return kfn(user_ids, item_ids, user_emb, item_emb)


def kernel(user_ids, item_ids, user_emb, item_emb):
    return _run(user_ids.astype(jnp.int32), item_ids.astype(jnp.int32),
                user_emb, item_emb)


# speedup vs baseline: 1.4397x; 1.0606x over previous
import functools

import jax
import jax.numpy as jnp
from jax import lax
from jax.experimental import pallas as pl
from jax.experimental.pallas import tpu as pltpu
from jax.experimental.pallas import tpu_sc as plsc

BATCH = 16384
EMB_DIM = 32
LANES = 16
ROWG = 8
NUM_WORKERS = 32
B_PER_W = BATCH // NUM_WORKERS
N_CHUNKS = B_PER_W // LANES


def _dot_sigmoid_kernel(user_ids_hbm, item_ids_hbm, user_emb_hbm,
                        item_emb_hbm, out_hbm,
                        uidx_v, iidx_v, ubuf_v, ibuf_v,
                        out_v, sem_u, sem_i):
    nc = 2
    wid = lax.axis_index("s") * nc + lax.axis_index("c")
    base = wid * B_PER_W

    pltpu.sync_copy(user_ids_hbm.at[pl.ds(base, B_PER_W)], uidx_v)
    pltpu.sync_copy(item_ids_hbm.at[pl.ds(base, B_PER_W)], iidx_v)
    lane_iota = lax.iota(jnp.int32, LANES)

    def fire(c, buf):
        off = c * LANES
        uvec = uidx_v[pl.ds(off, LANES)]
        ivec = iidx_v[pl.ds(off, LANES)]
        ubase = uvec & ~(ROWG - 1)
        ibase = ivec & ~(ROWG - 1)
        for j in range(LANES):
            urow = pl.multiple_of(ubase[j], ROWG)
            irow = pl.multiple_of(ibase[j], ROWG)
            pltpu.make_async_copy(
                user_emb_hbm.at[pl.ds(urow, ROWG), :],
                ubuf_v.at[buf, pl.ds(j * ROWG, ROWG), :], sem_u.at[buf]).start()
            pltpu.make_async_copy(
                item_emb_hbm.at[pl.ds(irow, ROWG), :],
                ibuf_v.at[buf, pl.ds(j * ROWG, ROWG), :], sem_i.at[buf]).start()

    def drain_compute(c, buf):
        off = c * LANES
        uvec = uidx_v[pl.ds(off, LANES)]
        ivec = iidx_v[pl.ds(off, LANES)]
        for j in range(LANES):
            pltpu.make_async_copy(
                user_emb_hbm.at[pl.ds(0, ROWG), :],
                ubuf_v.at[buf, pl.ds(j * ROWG, ROWG), :], sem_u.at[buf]).wait()
            pltpu.make_async_copy(
                item_emb_hbm.at[pl.ds(0, ROWG), :],
                ibuf_v.at[buf, pl.ds(j * ROWG, ROWG), :], sem_i.at[buf]).wait()
        urow_idx = lane_iota * ROWG + (uvec & (ROWG - 1))
        irow_idx = lane_iota * ROWG + (ivec & (ROWG - 1))
        acc = jnp.zeros((LANES,), jnp.float32)
        for k in range(EMB_DIM):
            kcol = jnp.full((LANES,), k, jnp.int32)
            uv = plsc.load_gather(ubuf_v.at[buf], [urow_idx, kcol])
            iv = plsc.load_gather(ibuf_v.at[buf], [irow_idx, kcol])
            acc = acc + uv * iv
        sig = 1.0 / (1.0 + jnp.exp(-acc))
        out_v[pl.ds(off, LANES)] = sig

    fire(0, 0)

    def pair_body(p, carry):
        c0 = p * 2
        fire(c0 + 1, 1)
        drain_compute(c0, 0)
        fire(jnp.minimum(c0 + 2, N_CHUNKS - 1), 0)
        drain_compute(c0 + 1, 1)
        return carry

    lax.fori_loop(0, N_CHUNKS // 2, pair_body, 0)
    for j in range(LANES):
        pltpu.make_async_copy(
            user_emb_hbm.at[pl.ds(0, ROWG), :],
            ubuf_v.at[0, pl.ds(j * ROWG, ROWG), :], sem_u.at[0]).wait()
        pltpu.make_async_copy(
            item_emb_hbm.at[pl.ds(0, ROWG), :],
            ibuf_v.at[0, pl.ds(j * ROWG, ROWG), :], sem_i.at[0]).wait()

    pltpu.sync_copy(out_v, out_hbm.at[pl.ds(base, B_PER_W)])


@jax.jit
def _run(user_ids, item_ids, user_emb, item_emb):
    mesh = plsc.VectorSubcoreMesh(core_axis_name="c", subcore_axis_name="s")
    kfn = functools.partial(
        pl.kernel,
        mesh=mesh,
        out_type=jax.ShapeDtypeStruct((BATCH,), jnp.float32),
        scratch_types=[
            pltpu.VMEM((B_PER_W,), jnp.int32),
            pltpu.VMEM((B_PER_W,), jnp.int32),
            pltpu.VMEM((2, LANES * ROWG, EMB_DIM), jnp.float32),
            pltpu.VMEM((2, LANES * ROWG, EMB_DIM), jnp.float32),
            pltpu.VMEM((B_PER_W,), jnp.float32),
            pltpu.SemaphoreType.DMA((2,)),
            pltpu.SemaphoreType.DMA((2,)),
        ],
        compiler_params=pltpu.CompilerParams(needs_layout_passes=False),
    )(_dot_sigmoid_kernel)
    return kfn(user_ids, item_ids, user_emb, item_emb)


def kernel(user_ids, item_ids, user_emb, item_emb):
    return _run(user_ids.astype(jnp.int32), item_ids.astype(jnp.int32),
                user_emb, item_emb)
